# Initial kernel scaffold; baseline (speedup 1.0000x reference)
import jax
import jax.numpy as jnp
from jax.experimental import pallas as pl


def kernel(x, W, b, Wr, br):
    raise NotImplementedError("write your pallas kernel here")


# speedup vs baseline: 4.4122x; 4.4122x over previous
import functools

import jax
import jax.numpy as jnp
from jax import lax
from jax.experimental import pallas as pl
from jax.experimental.pallas import tpu as pltpu
from jax.experimental.pallas import tpu_sc as plsc

IN_DIM = 2048
N_EXPERTS = 4
CHANNELS = 8192
B = 4096
K_KEEP = CHANNELS // 4

NW = 32
ROWS_PER = B // NW


def _mm_body(x_ref, w_ref, b_ref, o_ref):
    o_ref[...] = lax.dot_general(
        x_ref[...], w_ref[...],
        dimension_numbers=(((1,), (1,)), ((), ())),
        preferred_element_type=jnp.float32,
    ) + b_ref[...]


def _matmul(x, Wc, bc):
    BM, BN = 512, 1024
    return pl.pallas_call(
        _mm_body,
        grid=(B // BM, CHANNELS // BN),
        in_specs=[
            pl.BlockSpec((BM, IN_DIM), lambda i, j: (i, 0)),
            pl.BlockSpec((BN, IN_DIM), lambda i, j: (j, 0)),
            pl.BlockSpec((1, BN), lambda i, j: (0, j)),
        ],
        out_specs=pl.BlockSpec((BM, BN), lambda i, j: (i, j)),
        out_shape=jax.ShapeDtypeStruct((B, CHANNELS), jnp.float32),
        compiler_params=pltpu.CompilerParams(
            dimension_semantics=("parallel", "parallel"),
        ),
    )(x, Wc, bc.reshape(1, CHANNELS))


_BR = 256


def _sel_body(s_ref, thr_ref, cst_ref):
    key = lax.bitcast_convert_type(jnp.abs(s_ref[...]), jnp.int32)
    thr = jnp.zeros((_BR, 1), jnp.int32)
    for bit in range(30, -1, -1):
        cand = thr | (1 << bit)
        cnt = jnp.sum((key >= cand).astype(jnp.int32), axis=1, keepdims=True)
        thr = jnp.where(cnt >= K_KEEP, cand, thr)
    eq = key == thr
    n_gt = jnp.sum((key > thr).astype(jnp.int32), axis=1, keepdims=True)
    t_need = K_KEEP - n_gt
    col = lax.broadcasted_iota(jnp.int32, (_BR, CHANNELS), 1)
    cst = jnp.zeros((_BR, 1), jnp.int32)
    for bit in range(12, -1, -1):
        cand = cst | (1 << bit)
        cnt = jnp.sum((eq & (col >= cand)).astype(jnp.int32), axis=1,
                      keepdims=True)
        cst = jnp.where(cnt >= t_need, cand, cst)
    thr_f = lax.bitcast_convert_type(thr, jnp.float32)
    thr_ref[...] = jnp.broadcast_to(thr_f, (_BR, 128))
    cst_ref[...] = jnp.broadcast_to(cst, (_BR, 128))


def _select(scores):
    return pl.pallas_call(
        _sel_body,
        grid=(B // _BR,),
        in_specs=[pl.BlockSpec((_BR, CHANNELS), lambda i: (i, 0))],
        out_specs=[
            pl.BlockSpec((_BR, 128), lambda i: (i, 0)),
            pl.BlockSpec((_BR, 128), lambda i: (i, 0)),
        ],
        out_shape=[
            jax.ShapeDtypeStruct((B, 128), jnp.float32),
            jax.ShapeDtypeStruct((B, 128), jnp.int32),
        ],
        compiler_params=pltpu.CompilerParams(
            dimension_semantics=("arbitrary",),
        ),
    )(scores)


def _sc_body(ol_hbm, sc_hbm, thr_hbm, cst_hbm, out_hbm,
             vals_v, scor_v, orow_v, thr_v, cst_v):
    cid = lax.axis_index("c")
    sid = lax.axis_index("s")
    wid = sid * 2 + cid
    base = wid * ROWS_PER
    pltpu.sync_copy(thr_hbm.at[pl.ds(base * 128, ROWS_PER * 128)], thr_v)
    pltpu.sync_copy(cst_hbm.at[pl.ds(base * 128, ROWS_PER * 128)], cst_v)
    lane = lax.iota(jnp.int32, 16)
    v16 = jnp.full((16,), 16, jnp.int32)
    vm1 = jnp.full((16,), -1, jnp.int32)
    v1 = jnp.full((16,), 1, jnp.int32)
    v0 = jnp.zeros((16,), jnp.int32)
    c15 = jnp.reshape(jnp.full((16,), 15, jnp.int32), (16, 1))
    shuf = []
    for d in (1, 2, 4, 8):
        dv = jnp.full((16,), d, jnp.int32)
        idx_d = jnp.reshape(jnp.maximum(lane - dv, v0), (16, 1))
        shuf.append((idx_d, lane >= dv))
    _gdn = lax.GatherDimensionNumbers(
        offset_dims=(), collapsed_slice_dims=(0,), start_index_map=(0,))

    def _take16(v, idxcol):
        return lax.gather(v, idxcol, _gdn, slice_sizes=(1,),
                          mode=lax.GatherScatterMode.PROMISE_IN_BOUNDS)

    def _prefix_incl(ki):
        s = ki
        for idx_d, m_d in shuf:
            s = s + lax.select(m_d, _take16(s, idx_d), v0)
        return s

    def row_body(r, carry):
        row = base + r
        pltpu.sync_copy(ol_hbm.at[pl.ds(row * CHANNELS, CHANNELS)], vals_v)
        pltpu.sync_copy(sc_hbm.at[pl.ds(row * CHANNELS, CHANNELS)], scor_v)
        thr_b = thr_v[pl.ds(r * 128, 16)]
        cst_b = cst_v[pl.ds(r * 128, 16)]

        def blk_body(cb, carry2):
            off, cvec = carry2
            sa = jnp.abs(scor_v[pl.ds(cb * 16, 16)])
            keep = (sa > thr_b) | ((sa == thr_b) & (cvec >= cst_b))
            s = _prefix_incl(lax.select(keep, v1, v0))
            pos = s + vm1
            v = vals_v[pl.ds(cb * 16, 16)]
            plsc.store_scatter(orow_v, [off + pos], v, mask=keep)
            cnt = _take16(s, c15)
            return (off + cnt, cvec + v16)

        lax.fori_loop(0, CHANNELS // 16, blk_body,
                      (jnp.zeros((16,), jnp.int32), lane))
        pltpu.sync_copy(orow_v, out_hbm.at[pl.ds(row * K_KEEP, K_KEEP)])
        return carry

    lax.fori_loop(0, ROWS_PER, row_body, 0)


@functools.partial(
    pl.kernel,
    out_type=jax.ShapeDtypeStruct((B * K_KEEP,), jnp.float32),
    mesh=plsc.VectorSubcoreMesh(core_axis_name="c", subcore_axis_name="s"),
    compiler_params=pltpu.CompilerParams(needs_layout_passes=False),
    scratch_types=[
        pltpu.VMEM((CHANNELS,), jnp.float32),
        pltpu.VMEM((CHANNELS,), jnp.float32),
        pltpu.VMEM((K_KEEP,), jnp.float32),
        pltpu.VMEM((ROWS_PER * 128,), jnp.float32),
        pltpu.VMEM((ROWS_PER * 128,), jnp.int32),
    ],
)
def _sc_compact(ol_hbm, sc_hbm, thr_hbm, cst_hbm, out_hbm,
                vals_v, scor_v, orow_v, thr_v, cst_v):
    _sc_body(ol_hbm, sc_hbm, thr_hbm, cst_hbm, out_hbm,
             vals_v, scor_v, orow_v, thr_v, cst_v)


def kernel(x, W, b, Wr, br):
    scores = x @ Wr.T + br
    out_list = _matmul(x, W.reshape(CHANNELS, IN_DIM), b.reshape(CHANNELS))
    thr, cst = _select(scores)
    out = _sc_compact(out_list.reshape(B * CHANNELS),
                      scores.reshape(B * CHANNELS),
                      thr.reshape(B * 128),
                      cst.reshape(B * 128))
    return out.reshape(B, K_KEEP)


# speedup vs baseline: 4.8009x; 1.0881x over previous
import functools

import jax
import jax.numpy as jnp
from jax import lax
from jax.experimental import pallas as pl
from jax.experimental.pallas import tpu as pltpu
from jax.experimental.pallas import tpu_sc as plsc

IN_DIM = 2048
N_EXPERTS = 4
CHANNELS = 8192
B = 4096
K_KEEP = CHANNELS // 4

NW = 32
ROWS_PER = B // NW


def _mm_body(x_ref, w_ref, b_ref, o_ref):
    o_ref[...] = lax.dot_general(
        x_ref[...], w_ref[...],
        dimension_numbers=(((1,), (1,)), ((), ())),
        preferred_element_type=jnp.float32,
    ) + b_ref[...]


def _matmul(x, Wc, bc):
    BM, BN = 512, 1024
    return pl.pallas_call(
        _mm_body,
        grid=(B // BM, CHANNELS // BN),
        in_specs=[
            pl.BlockSpec((BM, IN_DIM), lambda i, j: (i, 0)),
            pl.BlockSpec((BN, IN_DIM), lambda i, j: (j, 0)),
            pl.BlockSpec((1, BN), lambda i, j: (0, j)),
        ],
        out_specs=pl.BlockSpec((BM, BN), lambda i, j: (i, j)),
        out_shape=jax.ShapeDtypeStruct((B, CHANNELS), jnp.float32),
        compiler_params=pltpu.CompilerParams(
            dimension_semantics=("parallel", "parallel"),
        ),
    )(x, Wc, bc.reshape(1, CHANNELS))


_BR = 256


def _sel_body(s_ref, thr_ref, cst_ref):
    key = lax.bitcast_convert_type(jnp.abs(s_ref[...]), jnp.int32)
    thr = jnp.zeros((_BR, 1), jnp.int32)
    for bit in range(30, -1, -1):
        cand = thr | (1 << bit)
        cnt = jnp.sum((key >= cand).astype(jnp.int32), axis=1, keepdims=True)
        thr = jnp.where(cnt >= K_KEEP, cand, thr)
    eq = key == thr
    n_gt = jnp.sum((key > thr).astype(jnp.int32), axis=1, keepdims=True)
    t_need = K_KEEP - n_gt
    col = lax.broadcasted_iota(jnp.int32, (_BR, CHANNELS), 1)
    cst = jnp.zeros((_BR, 1), jnp.int32)
    for bit in range(12, -1, -1):
        cand = cst | (1 << bit)
        cnt = jnp.sum((eq & (col >= cand)).astype(jnp.int32), axis=1,
                      keepdims=True)
        cst = jnp.where(cnt >= t_need, cand, cst)
    thr_f = lax.bitcast_convert_type(thr, jnp.float32)
    thr_ref[...] = jnp.broadcast_to(thr_f, (_BR, 128))
    cst_ref[...] = jnp.broadcast_to(cst, (_BR, 128))


def _select(scores):
    return pl.pallas_call(
        _sel_body,
        grid=(B // _BR,),
        in_specs=[pl.BlockSpec((_BR, CHANNELS), lambda i: (i, 0))],
        out_specs=[
            pl.BlockSpec((_BR, 128), lambda i: (i, 0)),
            pl.BlockSpec((_BR, 128), lambda i: (i, 0)),
        ],
        out_shape=[
            jax.ShapeDtypeStruct((B, 128), jnp.float32),
            jax.ShapeDtypeStruct((B, 128), jnp.int32),
        ],
        compiler_params=pltpu.CompilerParams(
            dimension_semantics=("arbitrary",),
        ),
    )(scores)


def _sc_body(ol_hbm, sc_hbm, thr_hbm, cst_hbm, out_hbm,
             vals_v, scor_v, vals2_v, scor2_v, orow_v, thr_v, cst_v,
             sem0a, sem0b, sem1a, sem1b):
    cid = lax.axis_index("c")
    sid = lax.axis_index("s")
    wid = sid * 2 + cid
    base = wid * ROWS_PER
    pltpu.sync_copy(thr_hbm.at[pl.ds(base * 128, ROWS_PER * 128)], thr_v)
    pltpu.sync_copy(cst_hbm.at[pl.ds(base * 128, ROWS_PER * 128)], cst_v)
    lane = lax.iota(jnp.int32, 16)
    v16 = jnp.full((16,), 16, jnp.int32)
    vm1 = jnp.full((16,), -1, jnp.int32)
    v1 = jnp.full((16,), 1, jnp.int32)
    v0 = jnp.zeros((16,), jnp.int32)
    c15 = jnp.reshape(jnp.full((16,), 15, jnp.int32), (16, 1))
    shuf = []
    for d in (1, 2, 4, 8):
        dv = jnp.full((16,), d, jnp.int32)
        idx_d = jnp.reshape(jnp.maximum(lane - dv, v0), (16, 1))
        shuf.append((idx_d, lane >= dv))
    _gdn = lax.GatherDimensionNumbers(
        offset_dims=(), collapsed_slice_dims=(0,), start_index_map=(0,))

    def _take16(v, idxcol):
        return lax.gather(v, idxcol, _gdn, slice_sizes=(1,),
                          mode=lax.GatherScatterMode.PROMISE_IN_BOUNDS)

    def _prefix_incl(ki):
        s = ki
        for idx_d, m_d in shuf:
            s = s + lax.select(m_d, _take16(s, idx_d), v0)
        return s

    def _compact_row(r, vals, scor):
        thr_b = thr_v[pl.ds(r * 128, 16)]
        cst_b = cst_v[pl.ds(r * 128, 16)]

        def blk_body(cb, carry2):
            off, cvec = carry2
            sa = jnp.abs(scor[pl.ds(cb * 16, 16)])
            keep = (sa > thr_b) | ((sa == thr_b) & (cvec >= cst_b))
            s = _prefix_incl(lax.select(keep, v1, v0))
            pos = s + vm1
            v = vals[pl.ds(cb * 16, 16)]
            plsc.store_scatter(orow_v, [off + pos], v, mask=keep)
            cnt = _take16(s, c15)
            return (off + cnt, cvec + v16)

        lax.fori_loop(0, CHANNELS // 16, blk_body,
                      (jnp.zeros((16,), jnp.int32), lane))
        pltpu.sync_copy(orow_v,
                        out_hbm.at[pl.ds((base + r) * K_KEEP, K_KEEP)])

    def _start(r, vals, scor, sem_a, sem_b):
        row = base + r
        pltpu.async_copy(ol_hbm.at[pl.ds(row * CHANNELS, CHANNELS)], vals,
                         sem_a)
        pltpu.async_copy(sc_hbm.at[pl.ds(row * CHANNELS, CHANNELS)], scor,
                         sem_b)

    def _wait(vals, scor, sem_a, sem_b):
        pltpu.make_async_copy(ol_hbm.at[pl.ds(0, CHANNELS)], vals,
                              sem_a).wait()
        pltpu.make_async_copy(sc_hbm.at[pl.ds(0, CHANNELS)], scor,
                              sem_b).wait()

    _start(0, vals_v, scor_v, sem0a, sem0b)

    def pair_body(p, carry):
        r0 = 2 * p
        _start(r0 + 1, vals2_v, scor2_v, sem1a, sem1b)
        _wait(vals_v, scor_v, sem0a, sem0b)
        _compact_row(r0, vals_v, scor_v)

        @pl.when(p < ROWS_PER // 2 - 1)
        def _():
            _start(r0 + 2, vals_v, scor_v, sem0a, sem0b)

        _wait(vals2_v, scor2_v, sem1a, sem1b)
        _compact_row(r0 + 1, vals2_v, scor2_v)
        return carry

    lax.fori_loop(0, ROWS_PER // 2, pair_body, 0)


@functools.partial(
    pl.kernel,
    out_type=jax.ShapeDtypeStruct((B * K_KEEP,), jnp.float32),
    mesh=plsc.VectorSubcoreMesh(core_axis_name="c", subcore_axis_name="s"),
    compiler_params=pltpu.CompilerParams(needs_layout_passes=False),
    scratch_types=[
        pltpu.VMEM((CHANNELS,), jnp.float32),
        pltpu.VMEM((CHANNELS,), jnp.float32),
        pltpu.VMEM((CHANNELS,), jnp.float32),
        pltpu.VMEM((CHANNELS,), jnp.float32),
        pltpu.VMEM((K_KEEP,), jnp.float32),
        pltpu.VMEM((ROWS_PER * 128,), jnp.float32),
        pltpu.VMEM((ROWS_PER * 128,), jnp.int32),
        pltpu.SemaphoreType.DMA,
        pltpu.SemaphoreType.DMA,
        pltpu.SemaphoreType.DMA,
        pltpu.SemaphoreType.DMA,
    ],
)
def _sc_compact(ol_hbm, sc_hbm, thr_hbm, cst_hbm, out_hbm,
                vals_v, scor_v, vals2_v, scor2_v, orow_v, thr_v, cst_v,
                sem0a, sem0b, sem1a, sem1b):
    _sc_body(ol_hbm, sc_hbm, thr_hbm, cst_hbm, out_hbm,
             vals_v, scor_v, vals2_v, scor2_v, orow_v, thr_v, cst_v,
             sem0a, sem0b, sem1a, sem1b)


def kernel(x, W, b, Wr, br):
    scores = x @ Wr.T + br
    out_list = _matmul(x, W.reshape(CHANNELS, IN_DIM), b.reshape(CHANNELS))
    thr, cst = _select(scores)
    out = _sc_compact(out_list.reshape(B * CHANNELS),
                      scores.reshape(B * CHANNELS),
                      thr.reshape(B * 128),
                      cst.reshape(B * 128))
    return out.reshape(B, K_KEEP)


# speedup vs baseline: 4.9205x; 1.0249x over previous
import functools

import jax
import jax.numpy as jnp
from jax import lax
from jax.experimental import pallas as pl
from jax.experimental.pallas import tpu as pltpu
from jax.experimental.pallas import tpu_sc as plsc

IN_DIM = 2048
N_EXPERTS = 4
CHANNELS = 8192
B = 4096
K_KEEP = CHANNELS // 4

NW = 32
ROWS_PER = B // NW


def _mm_body(x_ref, w_ref, b_ref, o_ref):
    o_ref[...] = lax.dot_general(
        x_ref[...], w_ref[...],
        dimension_numbers=(((1,), (1,)), ((), ())),
        preferred_element_type=jnp.float32,
    ) + b_ref[...]


def _matmul(x, Wc, bc):
    BM, BN = 512, 1024
    return pl.pallas_call(
        _mm_body,
        grid=(B // BM, CHANNELS // BN),
        in_specs=[
            pl.BlockSpec((BM, IN_DIM), lambda i, j: (i, 0)),
            pl.BlockSpec((BN, IN_DIM), lambda i, j: (j, 0)),
            pl.BlockSpec((1, BN), lambda i, j: (0, j)),
        ],
        out_specs=pl.BlockSpec((BM, BN), lambda i, j: (i, j)),
        out_shape=jax.ShapeDtypeStruct((B, CHANNELS), jnp.float32),
        compiler_params=pltpu.CompilerParams(
            dimension_semantics=("parallel", "parallel"),
        ),
    )(x, Wc, bc.reshape(1, CHANNELS))


_BR = 256


def _sel_body(s_ref, thr_ref, cst_ref):
    key = lax.bitcast_convert_type(jnp.abs(s_ref[...]), jnp.int32)
    thr = jnp.zeros((_BR, 1), jnp.int32)
    for bit in range(30, -1, -1):
        cand = thr | (1 << bit)
        cnt = jnp.sum((key >= cand).astype(jnp.int32), axis=1, keepdims=True)
        thr = jnp.where(cnt >= K_KEEP, cand, thr)
    eq = key == thr
    n_gt = jnp.sum((key > thr).astype(jnp.int32), axis=1, keepdims=True)
    t_need = K_KEEP - n_gt
    col = lax.broadcasted_iota(jnp.int32, (_BR, CHANNELS), 1)
    cst = jnp.zeros((_BR, 1), jnp.int32)
    for bit in range(12, -1, -1):
        cand = cst | (1 << bit)
        cnt = jnp.sum((eq & (col >= cand)).astype(jnp.int32), axis=1,
                      keepdims=True)
        cst = jnp.where(cnt >= t_need, cand, cst)
    thr_f = lax.bitcast_convert_type(thr, jnp.float32)
    thr_ref[...] = jnp.broadcast_to(thr_f, (_BR, 128))
    cst_ref[...] = jnp.broadcast_to(cst, (_BR, 128))


def _select(scores):
    return pl.pallas_call(
        _sel_body,
        grid=(B // _BR,),
        in_specs=[pl.BlockSpec((_BR, CHANNELS), lambda i: (i, 0))],
        out_specs=[
            pl.BlockSpec((_BR, 128), lambda i: (i, 0)),
            pl.BlockSpec((_BR, 128), lambda i: (i, 0)),
        ],
        out_shape=[
            jax.ShapeDtypeStruct((B, 128), jnp.float32),
            jax.ShapeDtypeStruct((B, 128), jnp.int32),
        ],
        compiler_params=pltpu.CompilerParams(
            dimension_semantics=("arbitrary",),
        ),
    )(scores)


def _sc_body(ol_hbm, sc_hbm, thr_hbm, cst_hbm, out_hbm,
             vals_v, scor_v, vals2_v, scor2_v, orow_v, thr_v, cst_v,
             sem0a, sem0b, sem1a, sem1b):
    cid = lax.axis_index("c")
    sid = lax.axis_index("s")
    wid = sid * 2 + cid
    base = wid * ROWS_PER
    pltpu.sync_copy(thr_hbm.at[pl.ds(base * 128, ROWS_PER * 128)], thr_v)
    pltpu.sync_copy(cst_hbm.at[pl.ds(base * 128, ROWS_PER * 128)], cst_v)
    lane = lax.iota(jnp.int32, 16)
    v16 = jnp.full((16,), 16, jnp.int32)
    vm1 = jnp.full((16,), -1, jnp.int32)
    v1 = jnp.full((16,), 1, jnp.int32)
    v0 = jnp.zeros((16,), jnp.int32)
    c15 = jnp.reshape(jnp.full((16,), 15, jnp.int32), (16, 1))
    shuf = []
    for d in (1, 2, 4, 8):
        dv = jnp.full((16,), d, jnp.int32)
        idx_d = jnp.reshape(jnp.maximum(lane - dv, v0), (16, 1))
        shuf.append((idx_d, lane >= dv))
    _gdn = lax.GatherDimensionNumbers(
        offset_dims=(), collapsed_slice_dims=(0,), start_index_map=(0,))

    def _take16(v, idxcol):
        return lax.gather(v, idxcol, _gdn, slice_sizes=(1,),
                          mode=lax.GatherScatterMode.PROMISE_IN_BOUNDS)

    def _prefix_incl(ki):
        s = ki
        for idx_d, m_d in shuf:
            s = s + lax.select(m_d, _take16(s, idx_d), v0)
        return s

    def _compact_row(r, vals, scor):
        thr_b = thr_v[pl.ds(r * 128, 16)]
        cst_b = cst_v[pl.ds(r * 128, 16)]

        def blk_body(cb, carry2):
            off, cvec = carry2
            sa = jnp.abs(scor[pl.ds(cb * 16, 16)])
            keep = (sa > thr_b) | ((sa == thr_b) & (cvec >= cst_b))
            s = _prefix_incl(lax.select(keep, v1, v0))
            pos = s + vm1
            v = vals[pl.ds(cb * 16, 16)]
            plsc.store_scatter(orow_v, [off + pos], v, mask=keep)
            cnt = _take16(s, c15)
            return (off + cnt, cvec + v16)

        lax.fori_loop(0, CHANNELS // 16, blk_body,
                      (jnp.zeros((16,), jnp.int32), lane), unroll=4)
        pltpu.sync_copy(orow_v,
                        out_hbm.at[pl.ds((base + r) * K_KEEP, K_KEEP)])

    def _start(r, vals, scor, sem_a, sem_b):
        row = base + r
        pltpu.async_copy(ol_hbm.at[pl.ds(row * CHANNELS, CHANNELS)], vals,
                         sem_a)
        pltpu.async_copy(sc_hbm.at[pl.ds(row * CHANNELS, CHANNELS)], scor,
                         sem_b)

    def _wait(vals, scor, sem_a, sem_b):
        pltpu.make_async_copy(ol_hbm.at[pl.ds(0, CHANNELS)], vals,
                              sem_a).wait()
        pltpu.make_async_copy(sc_hbm.at[pl.ds(0, CHANNELS)], scor,
                              sem_b).wait()

    _start(0, vals_v, scor_v, sem0a, sem0b)

    def pair_body(p, carry):
        r0 = 2 * p
        _start(r0 + 1, vals2_v, scor2_v, sem1a, sem1b)
        _wait(vals_v, scor_v, sem0a, sem0b)
        _compact_row(r0, vals_v, scor_v)

        @pl.when(p < ROWS_PER // 2 - 1)
        def _():
            _start(r0 + 2, vals_v, scor_v, sem0a, sem0b)

        _wait(vals2_v, scor2_v, sem1a, sem1b)
        _compact_row(r0 + 1, vals2_v, scor2_v)
        return carry

    lax.fori_loop(0, ROWS_PER // 2, pair_body, 0)


@functools.partial(
    pl.kernel,
    out_type=jax.ShapeDtypeStruct((B * K_KEEP,), jnp.float32),
    mesh=plsc.VectorSubcoreMesh(core_axis_name="c", subcore_axis_name="s"),
    compiler_params=pltpu.CompilerParams(needs_layout_passes=False),
    scratch_types=[
        pltpu.VMEM((CHANNELS,), jnp.float32),
        pltpu.VMEM((CHANNELS,), jnp.float32),
        pltpu.VMEM((CHANNELS,), jnp.float32),
        pltpu.VMEM((CHANNELS,), jnp.float32),
        pltpu.VMEM((K_KEEP,), jnp.float32),
        pltpu.VMEM((ROWS_PER * 128,), jnp.float32),
        pltpu.VMEM((ROWS_PER * 128,), jnp.int32),
        pltpu.SemaphoreType.DMA,
        pltpu.SemaphoreType.DMA,
        pltpu.SemaphoreType.DMA,
        pltpu.SemaphoreType.DMA,
    ],
)
def _sc_compact(ol_hbm, sc_hbm, thr_hbm, cst_hbm, out_hbm,
                vals_v, scor_v, vals2_v, scor2_v, orow_v, thr_v, cst_v,
                sem0a, sem0b, sem1a, sem1b):
    _sc_body(ol_hbm, sc_hbm, thr_hbm, cst_hbm, out_hbm,
             vals_v, scor_v, vals2_v, scor2_v, orow_v, thr_v, cst_v,
             sem0a, sem0b, sem1a, sem1b)


def kernel(x, W, b, Wr, br):
    scores = x @ Wr.T + br
    out_list = _matmul(x, W.reshape(CHANNELS, IN_DIM), b.reshape(CHANNELS))
    thr, cst = _select(scores)
    out = _sc_compact(out_list.reshape(B * CHANNELS),
                      scores.reshape(B * CHANNELS),
                      thr.reshape(B * 128),
                      cst.reshape(B * 128))
    return out.reshape(B, K_KEEP)


# speedup vs baseline: 4.9278x; 1.0015x over previous
import functools

import jax
import jax.numpy as jnp
from jax import lax
from jax.experimental import pallas as pl
from jax.experimental.pallas import tpu as pltpu
from jax.experimental.pallas import tpu_sc as plsc

IN_DIM = 2048
N_EXPERTS = 4
CHANNELS = 8192
B = 4096
K_KEEP = CHANNELS // 4

NW = 32
ROWS_PER = B // NW


def _mm_body(x_ref, w_ref, b_ref, o_ref):
    o_ref[...] = lax.dot_general(
        x_ref[...], w_ref[...],
        dimension_numbers=(((1,), (1,)), ((), ())),
        preferred_element_type=jnp.float32,
    ) + b_ref[...]


def _matmul(x, Wc, bc):
    BM, BN = 512, 1024
    return pl.pallas_call(
        _mm_body,
        grid=(B // BM, CHANNELS // BN),
        in_specs=[
            pl.BlockSpec((BM, IN_DIM), lambda i, j: (i, 0)),
            pl.BlockSpec((BN, IN_DIM), lambda i, j: (j, 0)),
            pl.BlockSpec((1, BN), lambda i, j: (0, j)),
        ],
        out_specs=pl.BlockSpec((BM, BN), lambda i, j: (i, j)),
        out_shape=jax.ShapeDtypeStruct((B, CHANNELS), jnp.float32),
        compiler_params=pltpu.CompilerParams(
            dimension_semantics=("parallel", "parallel"),
        ),
    )(x, Wc, bc.reshape(1, CHANNELS))


_BR = 256


def _sel_body(s_ref, thr_ref, cst_ref):
    key = lax.bitcast_convert_type(jnp.abs(s_ref[...]), jnp.int32)
    thr = jnp.zeros((_BR, 1), jnp.int32)
    for bit in range(30, -1, -1):
        cand = thr | (1 << bit)
        cnt = jnp.sum((key >= cand).astype(jnp.int32), axis=1, keepdims=True)
        thr = jnp.where(cnt >= K_KEEP, cand, thr)
    eq = key == thr
    n_gt = jnp.sum((key > thr).astype(jnp.int32), axis=1, keepdims=True)
    t_need = K_KEEP - n_gt
    col = lax.broadcasted_iota(jnp.int32, (_BR, CHANNELS), 1)
    cst = jnp.zeros((_BR, 1), jnp.int32)
    for bit in range(12, -1, -1):
        cand = cst | (1 << bit)
        cnt = jnp.sum((eq & (col >= cand)).astype(jnp.int32), axis=1,
                      keepdims=True)
        cst = jnp.where(cnt >= t_need, cand, cst)
    thr_f = lax.bitcast_convert_type(thr, jnp.float32)
    thr_ref[...] = jnp.broadcast_to(thr_f, (_BR, 128))
    cst_ref[...] = jnp.broadcast_to(cst, (_BR, 128))


def _select(scores):
    return pl.pallas_call(
        _sel_body,
        grid=(B // _BR,),
        in_specs=[pl.BlockSpec((_BR, CHANNELS), lambda i: (i, 0))],
        out_specs=[
            pl.BlockSpec((_BR, 128), lambda i: (i, 0)),
            pl.BlockSpec((_BR, 128), lambda i: (i, 0)),
        ],
        out_shape=[
            jax.ShapeDtypeStruct((B, 128), jnp.float32),
            jax.ShapeDtypeStruct((B, 128), jnp.int32),
        ],
        compiler_params=pltpu.CompilerParams(
            dimension_semantics=("arbitrary",),
        ),
    )(scores)


def _sc_body(ol_hbm, sc_hbm, thr_hbm, cst_hbm, out_hbm,
             vals_v, scor_v, vals2_v, scor2_v, orow_v, thr_v, cst_v,
             sem0a, sem0b, sem1a, sem1b):
    cid = lax.axis_index("c")
    sid = lax.axis_index("s")
    wid = sid * 2 + cid
    base = wid * ROWS_PER
    pltpu.sync_copy(thr_hbm.at[pl.ds(base * 128, ROWS_PER * 128)], thr_v)
    pltpu.sync_copy(cst_hbm.at[pl.ds(base * 128, ROWS_PER * 128)], cst_v)
    lane = lax.iota(jnp.int32, 16)
    v16 = jnp.full((16,), 16, jnp.int32)
    vm1 = jnp.full((16,), -1, jnp.int32)
    v1 = jnp.full((16,), 1, jnp.int32)
    v0 = jnp.zeros((16,), jnp.int32)
    c15 = jnp.reshape(jnp.full((16,), 15, jnp.int32), (16, 1))
    shuf = []
    for d in (1, 2, 4, 8):
        dv = jnp.full((16,), d, jnp.int32)
        idx_d = jnp.reshape(jnp.maximum(lane - dv, v0), (16, 1))
        shuf.append((idx_d, lane >= dv))
    _gdn = lax.GatherDimensionNumbers(
        offset_dims=(), collapsed_slice_dims=(0,), start_index_map=(0,))

    def _take16(v, idxcol):
        return lax.gather(v, idxcol, _gdn, slice_sizes=(1,),
                          mode=lax.GatherScatterMode.PROMISE_IN_BOUNDS)

    def _prefix_incl(ki):
        s = ki
        for idx_d, m_d in shuf:
            s = s + lax.select(m_d, _take16(s, idx_d), v0)
        return s

    def _compact_row(r, vals, scor):
        thr_b = thr_v[pl.ds(r * 128, 16)]
        cst_b = cst_v[pl.ds(r * 128, 16)]

        def blk_body(cb, carry2):
            off, cvec = carry2
            sa = jnp.abs(scor[pl.ds(cb * 16, 16)])
            keep = (sa > thr_b) | ((sa == thr_b) & (cvec >= cst_b))
            s = _prefix_incl(lax.select(keep, v1, v0))
            pos = s + vm1
            v = vals[pl.ds(cb * 16, 16)]
            plsc.store_scatter(orow_v, [off + pos], v, mask=keep)
            cnt = _take16(s, c15)
            return (off + cnt, cvec + v16)

        lax.fori_loop(0, CHANNELS // 16, blk_body,
                      (jnp.zeros((16,), jnp.int32), lane), unroll=8)
        pltpu.sync_copy(orow_v,
                        out_hbm.at[pl.ds((base + r) * K_KEEP, K_KEEP)])

    def _start(r, vals, scor, sem_a, sem_b):
        row = base + r
        pltpu.async_copy(ol_hbm.at[pl.ds(row * CHANNELS, CHANNELS)], vals,
                         sem_a)
        pltpu.async_copy(sc_hbm.at[pl.ds(row * CHANNELS, CHANNELS)], scor,
                         sem_b)

    def _wait(vals, scor, sem_a, sem_b):
        pltpu.make_async_copy(ol_hbm.at[pl.ds(0, CHANNELS)], vals,
                              sem_a).wait()
        pltpu.make_async_copy(sc_hbm.at[pl.ds(0, CHANNELS)], scor,
                              sem_b).wait()

    _start(0, vals_v, scor_v, sem0a, sem0b)

    def pair_body(p, carry):
        r0 = 2 * p
        _start(r0 + 1, vals2_v, scor2_v, sem1a, sem1b)
        _wait(vals_v, scor_v, sem0a, sem0b)
        _compact_row(r0, vals_v, scor_v)

        @pl.when(p < ROWS_PER // 2 - 1)
        def _():
            _start(r0 + 2, vals_v, scor_v, sem0a, sem0b)

        _wait(vals2_v, scor2_v, sem1a, sem1b)
        _compact_row(r0 + 1, vals2_v, scor2_v)
        return carry

    lax.fori_loop(0, ROWS_PER // 2, pair_body, 0)


@functools.partial(
    pl.kernel,
    out_type=jax.ShapeDtypeStruct((B * K_KEEP,), jnp.float32),
    mesh=plsc.VectorSubcoreMesh(core_axis_name="c", subcore_axis_name="s"),
    compiler_params=pltpu.CompilerParams(needs_layout_passes=False),
    scratch_types=[
        pltpu.VMEM((CHANNELS,), jnp.float32),
        pltpu.VMEM((CHANNELS,), jnp.float32),
        pltpu.VMEM((CHANNELS,), jnp.float32),
        pltpu.VMEM((CHANNELS,), jnp.float32),
        pltpu.VMEM((K_KEEP,), jnp.float32),
        pltpu.VMEM((ROWS_PER * 128,), jnp.float32),
        pltpu.VMEM((ROWS_PER * 128,), jnp.int32),
        pltpu.SemaphoreType.DMA,
        pltpu.SemaphoreType.DMA,
        pltpu.SemaphoreType.DMA,
        pltpu.SemaphoreType.DMA,
    ],
)
def _sc_compact(ol_hbm, sc_hbm, thr_hbm, cst_hbm, out_hbm,
                vals_v, scor_v, vals2_v, scor2_v, orow_v, thr_v, cst_v,
                sem0a, sem0b, sem1a, sem1b):
    _sc_body(ol_hbm, sc_hbm, thr_hbm, cst_hbm, out_hbm,
             vals_v, scor_v, vals2_v, scor2_v, orow_v, thr_v, cst_v,
             sem0a, sem0b, sem1a, sem1b)


def kernel(x, W, b, Wr, br):
    scores = x @ Wr.T + br
    out_list = _matmul(x, W.reshape(CHANNELS, IN_DIM), b.reshape(CHANNELS))
    thr, cst = _select(scores)
    out = _sc_compact(out_list.reshape(B * CHANNELS),
                      scores.reshape(B * CHANNELS),
                      thr.reshape(B * 128),
                      cst.reshape(B * 128))
    return out.reshape(B, K_KEEP)
